# Initial kernel scaffold; baseline (speedup 1.0000x reference)
import jax
import jax.numpy as jnp
from jax.experimental import pallas as pl


def kernel(x, edge_index, conv0_W, conv0_b, conv1_W, conv1_b, lin0_W, lin0_b, lin1_W, lin1_b, ln0_g, ln0_b, ln1_g, ln1_b, feat0, feat1, We0, We1, Be0, Be1, Le0, Le1, enc_W, enc_b, encln_g, encln_b):
    raise NotImplementedError("write your pallas kernel here")


# speedup vs baseline: 10.2517x; 10.2517x over previous
import functools

import jax
import jax.numpy as jnp
from jax import lax
from jax.experimental import pallas as pl
from jax.experimental.pallas import tpu as pltpu
from jax.experimental.pallas import tpu_sc as plsc

EPS = 1e-10
N = 10000
E = 320000
D = 128
H = 128

NC = 2
NS = 16
NW = NC * NS
EPW = E // NW
K = 80
CH = EPW // K
UNIT = 80
NU = N // UNIT
NUF = NU // NS
NUR = NU - NUF * NS
BLK = 1000
HIGH = lax.Precision.HIGHEST

@functools.cache
def _mesh():
    return plsc.VectorSubcoreMesh(core_axis_name="c", subcore_axis_name="s")


def _sc_degree(dst):
    @functools.partial(
        pl.kernel,
        out_type=jax.ShapeDtypeStruct((NC, N, D), jnp.float32),
        mesh=_mesh(),
        scratch_types=[
            pltpu.VMEM((K,), jnp.int32),
            pltpu.VMEM((K, D), jnp.float32),
            pltpu.VMEM((UNIT, D), jnp.float32),
            pltpu.VMEM_SHARED((N, D), jnp.float32),
            pltpu.SemaphoreType.DMA,
        ],
    )
    def k(dst_hbm, out_hbm, idx_v, ones_v, z_v, acc, sem):
        c = lax.axis_index("c")
        s = lax.axis_index("s")
        wid = c * NS + s

        @pl.loop(0, K)
        def _(r):
            @pl.loop(0, D, step=16)
            def _(cc):
                ones_v[r, pl.ds(cc, 16)] = jnp.ones((16,), jnp.float32)

        @pl.loop(0, UNIT)
        def _(r):
            @pl.loop(0, D, step=16)
            def _(cc):
                z_v[r, pl.ds(cc, 16)] = jnp.zeros((16,), jnp.float32)

        @pl.loop(0, NUF)
        def _(j):
            pltpu.sync_copy(z_v, acc.at[pl.ds((s + j * NS) * UNIT, UNIT)])

        @pl.when(s < NUR)
        def _():
            pltpu.sync_copy(z_v, acc.at[pl.ds((NUF * NS + s) * UNIT, UNIT)])

        plsc.subcore_barrier()

        base = wid * EPW

        @pl.loop(0, CH)
        def _(i):
            pltpu.sync_copy(dst_hbm.at[pl.ds(base + i * K, K)], idx_v)
            pltpu.sync_copy(ones_v, acc.at[idx_v], add=True)

        plsc.subcore_barrier()

        @pl.loop(0, NUF)
        def _(j):
            u = (s + j * NS) * UNIT
            pltpu.sync_copy(acc.at[pl.ds(u, UNIT)],
                            out_hbm.at[c, pl.ds(u, UNIT)])

        @pl.when(s < NUR)
        def _():
            u = (NUF * NS + s) * UNIT
            pltpu.sync_copy(acc.at[pl.ds(u, UNIT)],
                            out_hbm.at[c, pl.ds(u, UNIT)])

    return k(dst)


def _sc_segment_rows(hp, src, dst):
    @functools.partial(
        pl.kernel,
        out_type=jax.ShapeDtypeStruct((NC, N, D), jnp.float32),
        mesh=_mesh(),
        scratch_types=[
            pltpu.VMEM((K,), jnp.int32),
            pltpu.VMEM((K,), jnp.int32),
            pltpu.VMEM((K, D), jnp.float32),
            pltpu.VMEM((UNIT, D), jnp.float32),
            pltpu.VMEM_SHARED((N, D), jnp.float32),
            pltpu.SemaphoreType.DMA,
        ],
    )
    def k(hp_hbm, src_hbm, dst_hbm, out_hbm, src_v, dst_v, rows_v, z_v, acc, sem):
        c = lax.axis_index("c")
        s = lax.axis_index("s")
        wid = c * NS + s

        @pl.loop(0, UNIT)
        def _(r):
            @pl.loop(0, D, step=16)
            def _(cc):
                z_v[r, pl.ds(cc, 16)] = jnp.zeros((16,), jnp.float32)

        @pl.loop(0, NUF)
        def _(j):
            pltpu.sync_copy(z_v, acc.at[pl.ds((s + j * NS) * UNIT, UNIT)])

        @pl.when(s < NUR)
        def _():
            pltpu.sync_copy(z_v, acc.at[pl.ds((NUF * NS + s) * UNIT, UNIT)])

        plsc.subcore_barrier()

        base = wid * EPW

        @pl.loop(0, CH)
        def _(i):
            pltpu.sync_copy(src_hbm.at[pl.ds(base + i * K, K)], src_v)
            pltpu.sync_copy(dst_hbm.at[pl.ds(base + i * K, K)], dst_v)
            pltpu.async_copy(hp_hbm.at[src_v], rows_v, sem).wait()
            pltpu.sync_copy(rows_v, acc.at[dst_v], add=True)

        plsc.subcore_barrier()

        @pl.loop(0, NUF)
        def _(j):
            u = (s + j * NS) * UNIT
            pltpu.sync_copy(acc.at[pl.ds(u, UNIT)],
                            out_hbm.at[c, pl.ds(u, UNIT)])

        @pl.when(s < NUR)
        def _():
            u = (NUF * NS + s) * UNIT
            pltpu.sync_copy(acc.at[pl.ds(u, UNIT)],
                            out_hbm.at[c, pl.ds(u, UNIT)])

    return k(hp, src, dst)


def _deg_dinv(c_ref):
    cnt = c_ref[0, :, 0:1] + c_ref[1, :, 0:1]
    deg = cnt + 1.0
    return deg, lax.rsqrt(deg)


def _ln(t, g, b):
    m = jnp.mean(t, -1, keepdims=True)
    v = jnp.mean((t - m) ** 2, -1, keepdims=True)
    return (t - m) * lax.rsqrt(v + 1e-5) * g + b


def _tc_pre(x, W0, L0, counts, bsum):
    def body(x_ref, w_ref, l_ref, c_ref, b_ref, hp_ref, pre_ref):
        deg, dinv = _deg_dinv(c_ref)
        xv = x_ref[...]
        h0 = jnp.dot(xv, w_ref[...],
                     preferred_element_type=jnp.float32)
        r0 = jnp.dot(xv, l_ref[...],
                     preferred_element_type=jnp.float32)
        hp_ref[...] = h0 * dinv
        pre_ref[...] = h0 / deg + r0 + b_ref[...]

    return pl.pallas_call(
        body,
        grid=(N // BLK,),
        in_specs=[
            pl.BlockSpec((BLK, D), lambda i: (i, 0)),
            pl.BlockSpec((D, H), lambda i: (0, 0)),
            pl.BlockSpec((D, H), lambda i: (0, 0)),
            pl.BlockSpec((NC, BLK, D), lambda i: (0, i, 0)),
            pl.BlockSpec((1, H), lambda i: (0, 0)),
        ],
        out_specs=[
            pl.BlockSpec((BLK, H), lambda i: (i, 0)),
            pl.BlockSpec((BLK, H), lambda i: (i, 0)),
        ],
        out_shape=[jax.ShapeDtypeStruct((N, H), jnp.float32)] * 2,
    )(x, W0, L0, counts, bsum)


def _tc_layer(agg, counts, pre0, g0, b0, W1, L1, bsum):
    def body(a_ref, c_ref, p_ref, g_ref, b_ref, w_ref, l_ref, bs_ref,
             hp_ref, pre_ref):
        deg, dinv = _deg_dinv(c_ref)
        t = (a_ref[0] + a_ref[1]) * dinv + p_ref[...]
        h = jnp.maximum(_ln(t, g_ref[...], b_ref[...]), 0.0)
        h1 = jnp.dot(h, w_ref[...],
                     preferred_element_type=jnp.float32)
        r1 = jnp.dot(h, l_ref[...],
                     preferred_element_type=jnp.float32)
        hp_ref[...] = h1 * dinv
        pre_ref[...] = h1 / deg + r1 + bs_ref[...]

    return pl.pallas_call(
        body,
        grid=(N // BLK,),
        in_specs=[
            pl.BlockSpec((NC, BLK, H), lambda i: (0, i, 0)),
            pl.BlockSpec((NC, BLK, D), lambda i: (0, i, 0)),
            pl.BlockSpec((BLK, H), lambda i: (i, 0)),
            pl.BlockSpec((1, H), lambda i: (0, 0)),
            pl.BlockSpec((1, H), lambda i: (0, 0)),
            pl.BlockSpec((H, H), lambda i: (0, 0)),
            pl.BlockSpec((H, H), lambda i: (0, 0)),
            pl.BlockSpec((1, H), lambda i: (0, 0)),
        ],
        out_specs=[pl.BlockSpec((BLK, H), lambda i: (i, 0))] * 2,
        out_shape=[jax.ShapeDtypeStruct((N, H), jnp.float32)] * 2,
    )(agg, counts, pre0, g0, b0, W1, L1, bsum)


def _tc_fourier(feat0, We0, Be0, Le0, feat1, We1, Be1, Le1):
    def body(f0, w0, b0, l0, f1, w1, b1, l1, z0_ref, z1_ref):
        def four(f, w, bb, lam, kcurv):
            fv = f[...]
            wv = w[...]
            fn = jnp.sum(fv * fv, -1, keepdims=True)
            diff = fv[:, None, :] - wv[None, :, :]
            div = jnp.sum(diff * diff, axis=-1)
            dist = jnp.log((1.0 + kcurv * fn) / (div + EPS) + EPS)
            return jnp.exp((H - 1) * dist / 2.0) * jnp.cos(lam[...] * dist + bb[...])

        z0_ref[...] = four(f0, w0, b0, l0, -1.0)
        z1_ref[...] = four(f1, w1, b1, l1, 1.0)

    return pl.pallas_call(
        body,
        out_shape=[jax.ShapeDtypeStruct((H, H), jnp.float32)] * 2,
    )(feat0, We0, Be0, Le0, feat1, We1, Be1, Le1)


def _tc_final(agg, counts, pre1, g1, b1, z0, z1, enc_W, enc_b, eg, eb):
    def body(a_ref, c_ref, p_ref, g_ref, b_ref, z0_ref, z1_ref, w_ref, e_ref,
             eg_ref, eb_ref, o_ref):
        _, dinv = _deg_dinv(c_ref)
        t = (a_ref[0] + a_ref[1]) * dinv + p_ref[...]
        h2 = jnp.maximum(_ln(t, g_ref[...], b_ref[...]), 0.0)
        e0 = jnp.dot(h2, z0_ref[...], preferred_element_type=jnp.float32)
        e1 = jnp.dot(h2, z1_ref[...], preferred_element_type=jnp.float32)
        wv = w_ref[...]
        o = (jnp.dot(e0, wv[0:H, :], preferred_element_type=jnp.float32)
             + jnp.dot(e1, wv[H:2 * H, :], preferred_element_type=jnp.float32)
             + jnp.dot(h2, wv[2 * H:3 * H, :],
                       preferred_element_type=jnp.float32)
             + e_ref[...])
        o_ref[...] = _ln(o, eg_ref[...], eb_ref[...])

    return pl.pallas_call(
        body,
        grid=(N // BLK,),
        in_specs=[
            pl.BlockSpec((NC, BLK, H), lambda i: (0, i, 0)),
            pl.BlockSpec((NC, BLK, D), lambda i: (0, i, 0)),
            pl.BlockSpec((BLK, H), lambda i: (i, 0)),
            pl.BlockSpec((1, H), lambda i: (0, 0)),
            pl.BlockSpec((1, H), lambda i: (0, 0)),
            pl.BlockSpec((H, H), lambda i: (0, 0)),
            pl.BlockSpec((H, H), lambda i: (0, 0)),
            pl.BlockSpec((3 * H, H), lambda i: (0, 0)),
            pl.BlockSpec((1, H), lambda i: (0, 0)),
            pl.BlockSpec((1, H), lambda i: (0, 0)),
            pl.BlockSpec((1, H), lambda i: (0, 0)),
        ],
        out_specs=pl.BlockSpec((BLK, H), lambda i: (i, 0)),
        out_shape=jax.ShapeDtypeStruct((N, H), jnp.float32),
    )(agg, counts, pre1, g1, b1, z0, z1, enc_W, enc_b, eg, eb)


def kernel(x, edge_index, conv0_W, conv0_b, conv1_W, conv1_b, lin0_W, lin0_b,
           lin1_W, lin1_b, ln0_g, ln0_b, ln1_g, ln1_b, feat0, feat1, We0, We1,
           Be0, Be1, Le0, Le1, enc_W, enc_b, encln_g, encln_b):
    src = edge_index[0]
    dst = edge_index[1]
    row = lambda v: v.reshape(1, H)

    counts = _sc_degree(dst)
    z0, z1 = _tc_fourier(feat0, We0, Be0, Le0, feat1, We1, Be1, Le1)

    hp0, pre0 = _tc_pre(x, conv0_W, lin0_W, counts, row(conv0_b + lin0_b))
    agg0 = _sc_segment_rows(hp0, src, dst)
    hp1, pre1 = _tc_layer(agg0, counts, pre0, row(ln0_g), row(ln0_b),
                          conv1_W, lin1_W, row(conv1_b + lin1_b))
    agg1 = _sc_segment_rows(hp1, src, dst)
    return _tc_final(agg1, counts, pre1, row(ln1_g), row(ln1_b), z0, z1,
                     enc_W, row(enc_b), row(encln_g), row(encln_b))


# speedup vs baseline: 14.8613x; 1.4496x over previous
import functools

import jax
import jax.numpy as jnp
from jax import lax
from jax.experimental import pallas as pl
from jax.experimental.pallas import tpu as pltpu
from jax.experimental.pallas import tpu_sc as plsc

EPS = 1e-10
N = 10000
E = 320000
D = 128
H = 128

NC = 2
NS = 16
NW = NC * NS
EPW = E // NW
K = 80
CH = EPW // K
UNIT = 80
NU = N // UNIT
NUF = NU // NS
NUR = NU - NUF * NS
BLK = 1000
HIGH = lax.Precision.HIGHEST

@functools.cache
def _mesh():
    return plsc.VectorSubcoreMesh(core_axis_name="c", subcore_axis_name="s")


def _sc_degree(dst):
    @functools.partial(
        pl.kernel,
        out_type=jax.ShapeDtypeStruct((NC, N, D), jnp.float32),
        mesh=_mesh(),
        scratch_types=[
            pltpu.VMEM((K,), jnp.int32),
            pltpu.VMEM((K, D), jnp.float32),
            pltpu.VMEM((UNIT, D), jnp.float32),
            pltpu.VMEM_SHARED((N, D), jnp.float32),
            pltpu.SemaphoreType.DMA,
        ],
    )
    def k(dst_hbm, out_hbm, idx_v, ones_v, z_v, acc, sem):
        c = lax.axis_index("c")
        s = lax.axis_index("s")
        wid = c * NS + s

        @pl.loop(0, K)
        def _(r):
            @pl.loop(0, D, step=16)
            def _(cc):
                ones_v[r, pl.ds(cc, 16)] = jnp.ones((16,), jnp.float32)

        @pl.loop(0, UNIT)
        def _(r):
            @pl.loop(0, D, step=16)
            def _(cc):
                z_v[r, pl.ds(cc, 16)] = jnp.zeros((16,), jnp.float32)

        @pl.loop(0, NUF)
        def _(j):
            pltpu.sync_copy(z_v, acc.at[pl.ds((s + j * NS) * UNIT, UNIT)])

        @pl.when(s < NUR)
        def _():
            pltpu.sync_copy(z_v, acc.at[pl.ds((NUF * NS + s) * UNIT, UNIT)])

        plsc.subcore_barrier()

        base = wid * EPW

        @pl.loop(0, CH)
        def _(i):
            pltpu.sync_copy(dst_hbm.at[pl.ds(base + i * K, K)], idx_v)
            pltpu.sync_copy(ones_v, acc.at[idx_v], add=True)

        plsc.subcore_barrier()

        @pl.loop(0, NUF)
        def _(j):
            u = (s + j * NS) * UNIT
            pltpu.sync_copy(acc.at[pl.ds(u, UNIT)],
                            out_hbm.at[c, pl.ds(u, UNIT)])

        @pl.when(s < NUR)
        def _():
            u = (NUF * NS + s) * UNIT
            pltpu.sync_copy(acc.at[pl.ds(u, UNIT)],
                            out_hbm.at[c, pl.ds(u, UNIT)])

    return k(dst)


def _sc_segment_rows(hp, src, dst):
    @functools.partial(
        pl.kernel,
        out_type=jax.ShapeDtypeStruct((NC, N, D), jnp.float32),
        mesh=_mesh(),
        scratch_types=[
            pltpu.VMEM((K,), jnp.int32),
            pltpu.VMEM((K,), jnp.int32),
            pltpu.VMEM((K,), jnp.int32),
            pltpu.VMEM((K,), jnp.int32),
            pltpu.VMEM((K, D), jnp.float32),
            pltpu.VMEM((K, D), jnp.float32),
            pltpu.VMEM((UNIT, D), jnp.float32),
            pltpu.VMEM_SHARED((N, D), jnp.float32),
            pltpu.SemaphoreType.DMA,
            pltpu.SemaphoreType.DMA,
        ],
    )
    def k(hp_hbm, src_hbm, dst_hbm, out_hbm, src0, dst0, src1, dst1,
          rows0, rows1, z_v, acc, sem0, sem1):
        c = lax.axis_index("c")
        s = lax.axis_index("s")
        wid = c * NS + s

        @pl.loop(0, UNIT)
        def _(r):
            @pl.loop(0, D, step=16)
            def _(cc):
                z_v[r, pl.ds(cc, 16)] = jnp.zeros((16,), jnp.float32)

        @pl.loop(0, NUF)
        def _(j):
            pltpu.sync_copy(z_v, acc.at[pl.ds((s + j * NS) * UNIT, UNIT)])

        @pl.when(s < NUR)
        def _():
            pltpu.sync_copy(z_v, acc.at[pl.ds((NUF * NS + s) * UNIT, UNIT)])

        plsc.subcore_barrier()

        base = wid * EPW

        def fire(i, srcb, dstb, rowsb, sem):
            off = base + i * K
            pltpu.sync_copy(src_hbm.at[pl.ds(off, K)], srcb)
            pltpu.sync_copy(dst_hbm.at[pl.ds(off, K)], dstb)
            pltpu.async_copy(hp_hbm.at[srcb], rowsb, sem)

        def drain(srcb, rowsb, sem):
            pltpu.make_async_copy(hp_hbm.at[srcb], rowsb, sem).wait()

        fire(0, src0, dst0, rows0, sem0)

        @pl.loop(0, (CH + 1) // 2)
        def _(p):
            i0 = 2 * p

            @pl.when(i0 + 1 < CH)
            def _():
                fire(i0 + 1, src1, dst1, rows1, sem1)

            drain(src0, rows0, sem0)
            pltpu.sync_copy(rows0, acc.at[dst0], add=True)

            @pl.when(i0 + 2 < CH)
            def _():
                fire(i0 + 2, src0, dst0, rows0, sem0)

            @pl.when(i0 + 1 < CH)
            def _():
                drain(src1, rows1, sem1)
                pltpu.sync_copy(rows1, acc.at[dst1], add=True)

        plsc.subcore_barrier()

        @pl.loop(0, NUF)
        def _(j):
            u = (s + j * NS) * UNIT
            pltpu.sync_copy(acc.at[pl.ds(u, UNIT)],
                            out_hbm.at[c, pl.ds(u, UNIT)])

        @pl.when(s < NUR)
        def _():
            u = (NUF * NS + s) * UNIT
            pltpu.sync_copy(acc.at[pl.ds(u, UNIT)],
                            out_hbm.at[c, pl.ds(u, UNIT)])

    return k(hp, src, dst)


def _deg_dinv(c_ref):
    cnt = c_ref[0, :, 0:1] + c_ref[1, :, 0:1]
    deg = cnt + 1.0
    return deg, lax.rsqrt(deg)


def _ln(t, g, b):
    m = jnp.mean(t, -1, keepdims=True)
    v = jnp.mean((t - m) ** 2, -1, keepdims=True)
    return (t - m) * lax.rsqrt(v + 1e-5) * g + b


def _tc_pre(x, W0, L0, counts, bsum):
    def body(x_ref, w_ref, l_ref, c_ref, b_ref, hp_ref, pre_ref):
        deg, dinv = _deg_dinv(c_ref)
        xv = x_ref[...]
        h0 = jnp.dot(xv, w_ref[...],
                     preferred_element_type=jnp.float32)
        r0 = jnp.dot(xv, l_ref[...],
                     preferred_element_type=jnp.float32)
        hp_ref[...] = h0 * dinv
        pre_ref[...] = h0 / deg + r0 + b_ref[...]

    return pl.pallas_call(
        body,
        grid=(N // BLK,),
        in_specs=[
            pl.BlockSpec((BLK, D), lambda i: (i, 0)),
            pl.BlockSpec((D, H), lambda i: (0, 0)),
            pl.BlockSpec((D, H), lambda i: (0, 0)),
            pl.BlockSpec((NC, BLK, D), lambda i: (0, i, 0)),
            pl.BlockSpec((1, H), lambda i: (0, 0)),
        ],
        out_specs=[
            pl.BlockSpec((BLK, H), lambda i: (i, 0)),
            pl.BlockSpec((BLK, H), lambda i: (i, 0)),
        ],
        out_shape=[jax.ShapeDtypeStruct((N, H), jnp.float32)] * 2,
    )(x, W0, L0, counts, bsum)


def _tc_layer(agg, counts, pre0, g0, b0, W1, L1, bsum):
    def body(a_ref, c_ref, p_ref, g_ref, b_ref, w_ref, l_ref, bs_ref,
             hp_ref, pre_ref):
        deg, dinv = _deg_dinv(c_ref)
        t = (a_ref[0] + a_ref[1]) * dinv + p_ref[...]
        h = jnp.maximum(_ln(t, g_ref[...], b_ref[...]), 0.0)
        h1 = jnp.dot(h, w_ref[...],
                     preferred_element_type=jnp.float32)
        r1 = jnp.dot(h, l_ref[...],
                     preferred_element_type=jnp.float32)
        hp_ref[...] = h1 * dinv
        pre_ref[...] = h1 / deg + r1 + bs_ref[...]

    return pl.pallas_call(
        body,
        grid=(N // BLK,),
        in_specs=[
            pl.BlockSpec((NC, BLK, H), lambda i: (0, i, 0)),
            pl.BlockSpec((NC, BLK, D), lambda i: (0, i, 0)),
            pl.BlockSpec((BLK, H), lambda i: (i, 0)),
            pl.BlockSpec((1, H), lambda i: (0, 0)),
            pl.BlockSpec((1, H), lambda i: (0, 0)),
            pl.BlockSpec((H, H), lambda i: (0, 0)),
            pl.BlockSpec((H, H), lambda i: (0, 0)),
            pl.BlockSpec((1, H), lambda i: (0, 0)),
        ],
        out_specs=[pl.BlockSpec((BLK, H), lambda i: (i, 0))] * 2,
        out_shape=[jax.ShapeDtypeStruct((N, H), jnp.float32)] * 2,
    )(agg, counts, pre0, g0, b0, W1, L1, bsum)


def _tc_fourier(feat0, We0, Be0, Le0, feat1, We1, Be1, Le1):
    def body(f0, w0, b0, l0, f1, w1, b1, l1, z0_ref, z1_ref):
        def four(f, w, bb, lam, kcurv):
            fv = f[...]
            wv = w[...]
            fn = jnp.sum(fv * fv, -1, keepdims=True)
            diff = fv[:, None, :] - wv[None, :, :]
            div = jnp.sum(diff * diff, axis=-1)
            dist = jnp.log((1.0 + kcurv * fn) / (div + EPS) + EPS)
            return jnp.exp((H - 1) * dist / 2.0) * jnp.cos(lam[...] * dist + bb[...])

        z0_ref[...] = four(f0, w0, b0, l0, -1.0)
        z1_ref[...] = four(f1, w1, b1, l1, 1.0)

    return pl.pallas_call(
        body,
        out_shape=[jax.ShapeDtypeStruct((H, H), jnp.float32)] * 2,
    )(feat0, We0, Be0, Le0, feat1, We1, Be1, Le1)


def _tc_final(agg, counts, pre1, g1, b1, z0, z1, enc_W, enc_b, eg, eb):
    def body(a_ref, c_ref, p_ref, g_ref, b_ref, z0_ref, z1_ref, w_ref, e_ref,
             eg_ref, eb_ref, o_ref):
        _, dinv = _deg_dinv(c_ref)
        t = (a_ref[0] + a_ref[1]) * dinv + p_ref[...]
        h2 = jnp.maximum(_ln(t, g_ref[...], b_ref[...]), 0.0)
        e0 = jnp.dot(h2, z0_ref[...], preferred_element_type=jnp.float32)
        e1 = jnp.dot(h2, z1_ref[...], preferred_element_type=jnp.float32)
        wv = w_ref[...]
        o = (jnp.dot(e0, wv[0:H, :], preferred_element_type=jnp.float32)
             + jnp.dot(e1, wv[H:2 * H, :], preferred_element_type=jnp.float32)
             + jnp.dot(h2, wv[2 * H:3 * H, :],
                       preferred_element_type=jnp.float32)
             + e_ref[...])
        o_ref[...] = _ln(o, eg_ref[...], eb_ref[...])

    return pl.pallas_call(
        body,
        grid=(N // BLK,),
        in_specs=[
            pl.BlockSpec((NC, BLK, H), lambda i: (0, i, 0)),
            pl.BlockSpec((NC, BLK, D), lambda i: (0, i, 0)),
            pl.BlockSpec((BLK, H), lambda i: (i, 0)),
            pl.BlockSpec((1, H), lambda i: (0, 0)),
            pl.BlockSpec((1, H), lambda i: (0, 0)),
            pl.BlockSpec((H, H), lambda i: (0, 0)),
            pl.BlockSpec((H, H), lambda i: (0, 0)),
            pl.BlockSpec((3 * H, H), lambda i: (0, 0)),
            pl.BlockSpec((1, H), lambda i: (0, 0)),
            pl.BlockSpec((1, H), lambda i: (0, 0)),
            pl.BlockSpec((1, H), lambda i: (0, 0)),
        ],
        out_specs=pl.BlockSpec((BLK, H), lambda i: (i, 0)),
        out_shape=jax.ShapeDtypeStruct((N, H), jnp.float32),
    )(agg, counts, pre1, g1, b1, z0, z1, enc_W, enc_b, eg, eb)


def kernel(x, edge_index, conv0_W, conv0_b, conv1_W, conv1_b, lin0_W, lin0_b,
           lin1_W, lin1_b, ln0_g, ln0_b, ln1_g, ln1_b, feat0, feat1, We0, We1,
           Be0, Be1, Le0, Le1, enc_W, enc_b, encln_g, encln_b):
    src = edge_index[0]
    dst = edge_index[1]
    row = lambda v: v.reshape(1, H)

    counts = _sc_degree(dst)
    z0, z1 = _tc_fourier(feat0, We0, Be0, Le0, feat1, We1, Be1, Le1)

    hp0, pre0 = _tc_pre(x, conv0_W, lin0_W, counts, row(conv0_b + lin0_b))
    agg0 = _sc_segment_rows(hp0, src, dst)
    hp1, pre1 = _tc_layer(agg0, counts, pre0, row(ln0_g), row(ln0_b),
                          conv1_W, lin1_W, row(conv1_b + lin1_b))
    agg1 = _sc_segment_rows(hp1, src, dst)
    return _tc_final(agg1, counts, pre1, row(ln1_g), row(ln1_b), z0, z1,
                     enc_W, row(enc_b), row(encln_g), row(encln_b))


# speedup vs baseline: 16.3347x; 1.0991x over previous
import functools

import jax
import jax.numpy as jnp
from jax import lax
from jax.experimental import pallas as pl
from jax.experimental.pallas import tpu as pltpu
from jax.experimental.pallas import tpu_sc as plsc

EPS = 1e-10
N = 10000
E = 320000
D = 128
H = 128

NC = 2
NS = 16
NW = NC * NS
EPW = E // NW
K = 80
CH = EPW // K
UNIT = 80
NU = N // UNIT
NUF = NU // NS
NUR = NU - NUF * NS
BLK = 1000
HIGH = lax.Precision.HIGHEST

@functools.cache
def _mesh():
    return plsc.VectorSubcoreMesh(core_axis_name="c", subcore_axis_name="s")


def _sc_degree(dst):
    @functools.partial(
        pl.kernel,
        out_type=jax.ShapeDtypeStruct((NC, N, D), jnp.float32),
        mesh=_mesh(),
        scratch_types=[
            pltpu.VMEM((K,), jnp.int32),
            pltpu.VMEM((K,), jnp.int32),
            pltpu.VMEM((K, D), jnp.float32),
            pltpu.VMEM((UNIT, D), jnp.float32),
            pltpu.VMEM_SHARED((N, D), jnp.float32),
            pltpu.SemaphoreType.DMA,
            pltpu.SemaphoreType.DMA,
        ],
    )
    def k(dst_hbm, out_hbm, idx0, idx1, ones_v, z_v, acc, sem0, sem1):
        c = lax.axis_index("c")
        s = lax.axis_index("s")
        wid = c * NS + s

        @pl.loop(0, K)
        def _(r):
            @pl.loop(0, D, step=16)
            def _(cc):
                ones_v[r, pl.ds(cc, 16)] = jnp.ones((16,), jnp.float32)

        @pl.loop(0, UNIT)
        def _(r):
            @pl.loop(0, D, step=16)
            def _(cc):
                z_v[r, pl.ds(cc, 16)] = jnp.zeros((16,), jnp.float32)

        @pl.loop(0, NUF)
        def _(j):
            pltpu.sync_copy(z_v, acc.at[pl.ds((s + j * NS) * UNIT, UNIT)])

        @pl.when(s < NUR)
        def _():
            pltpu.sync_copy(z_v, acc.at[pl.ds((NUF * NS + s) * UNIT, UNIT)])

        plsc.subcore_barrier()

        base = wid * EPW

        def fire(i, buf, sem):
            pltpu.async_copy(dst_hbm.at[pl.ds(base + i * K, K)], buf, sem)

        def drain(i, buf, sem):
            pltpu.make_async_copy(dst_hbm.at[pl.ds(base + i * K, K)], buf,
                                  sem).wait()

        fire(0, idx0, sem0)

        @pl.loop(0, (CH + 1) // 2)
        def _(p):
            i0 = 2 * p

            @pl.when(i0 + 1 < CH)
            def _():
                fire(i0 + 1, idx1, sem1)

            drain(i0, idx0, sem0)
            pltpu.sync_copy(ones_v, acc.at[idx0], add=True)

            @pl.when(i0 + 2 < CH)
            def _():
                fire(i0 + 2, idx0, sem0)

            @pl.when(i0 + 1 < CH)
            def _():
                drain(i0 + 1, idx1, sem1)
                pltpu.sync_copy(ones_v, acc.at[idx1], add=True)

        plsc.subcore_barrier()

        @pl.loop(0, NUF)
        def _(j):
            u = (s + j * NS) * UNIT
            pltpu.sync_copy(acc.at[pl.ds(u, UNIT)],
                            out_hbm.at[c, pl.ds(u, UNIT)])

        @pl.when(s < NUR)
        def _():
            u = (NUF * NS + s) * UNIT
            pltpu.sync_copy(acc.at[pl.ds(u, UNIT)],
                            out_hbm.at[c, pl.ds(u, UNIT)])

    return k(dst)


def _sc_segment_rows(hp, src, dst):
    @functools.partial(
        pl.kernel,
        out_type=jax.ShapeDtypeStruct((NC, N, D), jnp.float32),
        mesh=_mesh(),
        scratch_types=[
            pltpu.VMEM((K,), jnp.int32),
            pltpu.VMEM((K,), jnp.int32),
            pltpu.VMEM((K,), jnp.int32),
            pltpu.VMEM((K,), jnp.int32),
            pltpu.VMEM((K, D), jnp.float32),
            pltpu.VMEM((K, D), jnp.float32),
            pltpu.VMEM((UNIT, D), jnp.float32),
            pltpu.VMEM_SHARED((N, D), jnp.float32),
            pltpu.SemaphoreType.DMA,
            pltpu.SemaphoreType.DMA,
        ],
    )
    def k(hp_hbm, src_hbm, dst_hbm, out_hbm, src0, dst0, src1, dst1,
          rows0, rows1, z_v, acc, sem0, sem1):
        c = lax.axis_index("c")
        s = lax.axis_index("s")
        wid = c * NS + s

        @pl.loop(0, UNIT)
        def _(r):
            @pl.loop(0, D, step=16)
            def _(cc):
                z_v[r, pl.ds(cc, 16)] = jnp.zeros((16,), jnp.float32)

        @pl.loop(0, NUF)
        def _(j):
            pltpu.sync_copy(z_v, acc.at[pl.ds((s + j * NS) * UNIT, UNIT)])

        @pl.when(s < NUR)
        def _():
            pltpu.sync_copy(z_v, acc.at[pl.ds((NUF * NS + s) * UNIT, UNIT)])

        plsc.subcore_barrier()

        base = wid * EPW

        def fire(i, srcb, dstb, rowsb, sem):
            off = base + i * K
            pltpu.sync_copy(src_hbm.at[pl.ds(off, K)], srcb)
            pltpu.sync_copy(dst_hbm.at[pl.ds(off, K)], dstb)
            pltpu.async_copy(hp_hbm.at[srcb], rowsb, sem)

        def drain(srcb, rowsb, sem):
            pltpu.make_async_copy(hp_hbm.at[srcb], rowsb, sem).wait()

        fire(0, src0, dst0, rows0, sem0)

        @pl.loop(0, (CH + 1) // 2)
        def _(p):
            i0 = 2 * p

            @pl.when(i0 + 1 < CH)
            def _():
                fire(i0 + 1, src1, dst1, rows1, sem1)

            drain(src0, rows0, sem0)
            pltpu.sync_copy(rows0, acc.at[dst0], add=True)

            @pl.when(i0 + 2 < CH)
            def _():
                fire(i0 + 2, src0, dst0, rows0, sem0)

            @pl.when(i0 + 1 < CH)
            def _():
                drain(src1, rows1, sem1)
                pltpu.sync_copy(rows1, acc.at[dst1], add=True)

        plsc.subcore_barrier()

        @pl.loop(0, NUF)
        def _(j):
            u = (s + j * NS) * UNIT
            pltpu.sync_copy(acc.at[pl.ds(u, UNIT)],
                            out_hbm.at[c, pl.ds(u, UNIT)])

        @pl.when(s < NUR)
        def _():
            u = (NUF * NS + s) * UNIT
            pltpu.sync_copy(acc.at[pl.ds(u, UNIT)],
                            out_hbm.at[c, pl.ds(u, UNIT)])

    return k(hp, src, dst)


def _deg_dinv(c_ref):
    cnt = c_ref[0, :, 0:1] + c_ref[1, :, 0:1]
    deg = cnt + 1.0
    return deg, lax.rsqrt(deg)


def _ln(t, g, b):
    m = jnp.mean(t, -1, keepdims=True)
    v = jnp.mean((t - m) ** 2, -1, keepdims=True)
    return (t - m) * lax.rsqrt(v + 1e-5) * g + b


def _tc_pre(x, W0, L0, counts, bsum):
    def body(x_ref, w_ref, l_ref, c_ref, b_ref, hp_ref, pre_ref):
        deg, dinv = _deg_dinv(c_ref)
        xv = x_ref[...]
        h0 = jnp.dot(xv, w_ref[...],
                     preferred_element_type=jnp.float32)
        r0 = jnp.dot(xv, l_ref[...],
                     preferred_element_type=jnp.float32)
        hp_ref[...] = h0 * dinv
        pre_ref[...] = h0 / deg + r0 + b_ref[...]

    return pl.pallas_call(
        body,
        grid=(N // BLK,),
        in_specs=[
            pl.BlockSpec((BLK, D), lambda i: (i, 0)),
            pl.BlockSpec((D, H), lambda i: (0, 0)),
            pl.BlockSpec((D, H), lambda i: (0, 0)),
            pl.BlockSpec((NC, BLK, D), lambda i: (0, i, 0)),
            pl.BlockSpec((1, H), lambda i: (0, 0)),
        ],
        out_specs=[
            pl.BlockSpec((BLK, H), lambda i: (i, 0)),
            pl.BlockSpec((BLK, H), lambda i: (i, 0)),
        ],
        out_shape=[jax.ShapeDtypeStruct((N, H), jnp.float32)] * 2,
    )(x, W0, L0, counts, bsum)


def _tc_layer(agg, counts, pre0, g0, b0, W1, L1, bsum):
    def body(a_ref, c_ref, p_ref, g_ref, b_ref, w_ref, l_ref, bs_ref,
             hp_ref, pre_ref):
        deg, dinv = _deg_dinv(c_ref)
        t = (a_ref[0] + a_ref[1]) * dinv + p_ref[...]
        h = jnp.maximum(_ln(t, g_ref[...], b_ref[...]), 0.0)
        h1 = jnp.dot(h, w_ref[...],
                     preferred_element_type=jnp.float32)
        r1 = jnp.dot(h, l_ref[...],
                     preferred_element_type=jnp.float32)
        hp_ref[...] = h1 * dinv
        pre_ref[...] = h1 / deg + r1 + bs_ref[...]

    return pl.pallas_call(
        body,
        grid=(N // BLK,),
        in_specs=[
            pl.BlockSpec((NC, BLK, H), lambda i: (0, i, 0)),
            pl.BlockSpec((NC, BLK, D), lambda i: (0, i, 0)),
            pl.BlockSpec((BLK, H), lambda i: (i, 0)),
            pl.BlockSpec((1, H), lambda i: (0, 0)),
            pl.BlockSpec((1, H), lambda i: (0, 0)),
            pl.BlockSpec((H, H), lambda i: (0, 0)),
            pl.BlockSpec((H, H), lambda i: (0, 0)),
            pl.BlockSpec((1, H), lambda i: (0, 0)),
        ],
        out_specs=[pl.BlockSpec((BLK, H), lambda i: (i, 0))] * 2,
        out_shape=[jax.ShapeDtypeStruct((N, H), jnp.float32)] * 2,
    )(agg, counts, pre0, g0, b0, W1, L1, bsum)


def _tc_fourier(feat0, We0, Be0, Le0, feat1, We1, Be1, Le1):
    def body(f0, w0, b0, l0, f1, w1, b1, l1, z0_ref, z1_ref):
        def four(f, w, bb, lam, kcurv):
            fv = f[...]
            wv = w[...]
            fn = jnp.sum(fv * fv, -1, keepdims=True)
            diff = fv[:, None, :] - wv[None, :, :]
            div = jnp.sum(diff * diff, axis=-1)
            dist = jnp.log((1.0 + kcurv * fn) / (div + EPS) + EPS)
            return jnp.exp((H - 1) * dist / 2.0) * jnp.cos(lam[...] * dist + bb[...])

        z0_ref[...] = four(f0, w0, b0, l0, -1.0)
        z1_ref[...] = four(f1, w1, b1, l1, 1.0)

    return pl.pallas_call(
        body,
        out_shape=[jax.ShapeDtypeStruct((H, H), jnp.float32)] * 2,
    )(feat0, We0, Be0, Le0, feat1, We1, Be1, Le1)


def _tc_final(agg, counts, pre1, g1, b1, z0, z1, enc_W, enc_b, eg, eb):
    def body(a_ref, c_ref, p_ref, g_ref, b_ref, z0_ref, z1_ref, w_ref, e_ref,
             eg_ref, eb_ref, o_ref):
        _, dinv = _deg_dinv(c_ref)
        t = (a_ref[0] + a_ref[1]) * dinv + p_ref[...]
        h2 = jnp.maximum(_ln(t, g_ref[...], b_ref[...]), 0.0)
        e0 = jnp.dot(h2, z0_ref[...], preferred_element_type=jnp.float32)
        e1 = jnp.dot(h2, z1_ref[...], preferred_element_type=jnp.float32)
        wv = w_ref[...]
        o = (jnp.dot(e0, wv[0:H, :], preferred_element_type=jnp.float32)
             + jnp.dot(e1, wv[H:2 * H, :], preferred_element_type=jnp.float32)
             + jnp.dot(h2, wv[2 * H:3 * H, :],
                       preferred_element_type=jnp.float32)
             + e_ref[...])
        o_ref[...] = _ln(o, eg_ref[...], eb_ref[...])

    return pl.pallas_call(
        body,
        grid=(N // BLK,),
        in_specs=[
            pl.BlockSpec((NC, BLK, H), lambda i: (0, i, 0)),
            pl.BlockSpec((NC, BLK, D), lambda i: (0, i, 0)),
            pl.BlockSpec((BLK, H), lambda i: (i, 0)),
            pl.BlockSpec((1, H), lambda i: (0, 0)),
            pl.BlockSpec((1, H), lambda i: (0, 0)),
            pl.BlockSpec((H, H), lambda i: (0, 0)),
            pl.BlockSpec((H, H), lambda i: (0, 0)),
            pl.BlockSpec((3 * H, H), lambda i: (0, 0)),
            pl.BlockSpec((1, H), lambda i: (0, 0)),
            pl.BlockSpec((1, H), lambda i: (0, 0)),
            pl.BlockSpec((1, H), lambda i: (0, 0)),
        ],
        out_specs=pl.BlockSpec((BLK, H), lambda i: (i, 0)),
        out_shape=jax.ShapeDtypeStruct((N, H), jnp.float32),
    )(agg, counts, pre1, g1, b1, z0, z1, enc_W, enc_b, eg, eb)


def kernel(x, edge_index, conv0_W, conv0_b, conv1_W, conv1_b, lin0_W, lin0_b,
           lin1_W, lin1_b, ln0_g, ln0_b, ln1_g, ln1_b, feat0, feat1, We0, We1,
           Be0, Be1, Le0, Le1, enc_W, enc_b, encln_g, encln_b):
    src = edge_index[0]
    dst = edge_index[1]
    row = lambda v: v.reshape(1, H)

    counts = _sc_degree(dst)
    z0, z1 = _tc_fourier(feat0, We0, Be0, Le0, feat1, We1, Be1, Le1)

    hp0, pre0 = _tc_pre(x, conv0_W, lin0_W, counts, row(conv0_b + lin0_b))
    agg0 = _sc_segment_rows(hp0, src, dst)
    hp1, pre1 = _tc_layer(agg0, counts, pre0, row(ln0_g), row(ln0_b),
                          conv1_W, lin1_W, row(conv1_b + lin1_b))
    agg1 = _sc_segment_rows(hp1, src, dst)
    return _tc_final(agg1, counts, pre1, row(ln1_g), row(ln1_b), z0, z1,
                     enc_W, row(enc_b), row(encln_g), row(encln_b))
